# baseline (speedup 1.0000x reference)
import jax
import jax.numpy as jnp
from jax.experimental import pallas as pl
from jax.experimental.pallas import tpu as pltpu

_N, _D, _E = 8192, 1024, 4
_BLK = 1024


def _fused_body(x_ref, wgn_ref, thr_ref, mask_ref, noise_ref,
                W1_ref, b1_ref, W2_ref, b2_ref, out_ref):
    x = x_ref[...]
    logits = jnp.dot(x, wgn_ref[...], preferred_element_type=jnp.float32)
    clean = logits[:, :_E]
    raw_noise = logits[:, _E:]
    noise_std = jax.nn.softplus(raw_noise) + 1e-2
    noisy = clean + noise_ref[...] * noise_std
    scores = noisy - thr_ref[...]
    signed = jnp.sign(scores)
    sel = 0.5 * (signed + 1.0) * mask_ref[...]
    masked = jnp.where(sel > 0.0, clean, jnp.full_like(clean, -1e9))
    m = jnp.max(masked, axis=-1, keepdims=True)
    ex = jnp.exp(masked - m)
    gates = (ex / jnp.sum(ex, axis=-1, keepdims=True)) * sel
    denom = jnp.clip(jnp.sum(gates, axis=-1, keepdims=True), 1e-9, None)
    gates = gates / denom

    acc = None
    for e in range(_E):
        h = jnp.dot(x, W1_ref[e], preferred_element_type=jnp.float32)
        h = jnp.maximum(h + b1_ref[e:e + 1, :], 0.0)
        y = jnp.dot(h, W2_ref[e], preferred_element_type=jnp.float32)
        y = y + b2_ref[e:e + 1, :]
        gy = gates[:, e:e + 1] * y
        acc = gy if acc is None else acc + gy
    out_ref[...] = acc


def kernel(x, w_gate, w_noise, gate_threshold, experts_mask, noise, W1, b1, W2, b2):
    wgn = jnp.concatenate([w_gate, w_noise], axis=1)
    thr = gate_threshold.reshape(1, _E)
    mask = experts_mask.reshape(1, _E)

    grid = (_N // _BLK,)
    out = pl.pallas_call(
        _fused_body,
        grid=grid,
        in_specs=[
            pl.BlockSpec((_BLK, _D), lambda i: (i, 0)),
            pl.BlockSpec((_D, 2 * _E), lambda i: (0, 0)),
            pl.BlockSpec((1, _E), lambda i: (0, 0)),
            pl.BlockSpec((1, _E), lambda i: (0, 0)),
            pl.BlockSpec((_BLK, _E), lambda i: (i, 0)),
            pl.BlockSpec((_E, _D, _D), lambda i: (0, 0, 0)),
            pl.BlockSpec((_E, _D), lambda i: (0, 0)),
            pl.BlockSpec((_E, _D, _D), lambda i: (0, 0, 0)),
            pl.BlockSpec((_E, _D), lambda i: (0, 0)),
        ],
        out_specs=pl.BlockSpec((_BLK, _D), lambda i: (i, 0)),
        out_shape=jax.ShapeDtypeStruct((_N, _D), jnp.float32),
        compiler_params=pltpu.CompilerParams(
            dimension_semantics=("arbitrary",),
            vmem_limit_bytes=100 * 1024 * 1024,
        ),
    )(x, wgn, thr, mask, noise, W1, b1, W2, b2)
    return out


# speedup vs baseline: 1.0017x; 1.0017x over previous
import jax
import jax.numpy as jnp
from jax.experimental import pallas as pl
from jax.experimental.pallas import tpu as pltpu

_N, _D, _E = 8192, 1024, 4
_BLK = 512


def _fused_body(x_ref, wgn_ref, thr_ref, mask_ref, noise_ref,
                W1_ref, b1_ref, W2_ref, b2_ref, out_ref):
    x = x_ref[...]
    logits = jnp.dot(x, wgn_ref[...], preferred_element_type=jnp.float32)
    clean = logits[:, :_E]
    raw_noise = logits[:, _E:]
    noise_std = jax.nn.softplus(raw_noise) + 1e-2
    noisy = clean + noise_ref[...] * noise_std
    scores = noisy - thr_ref[...]
    signed = jnp.sign(scores)
    sel = 0.5 * (signed + 1.0) * mask_ref[...]
    masked = jnp.where(sel > 0.0, clean, jnp.full_like(clean, -1e9))
    m = jnp.max(masked, axis=-1, keepdims=True)
    ex = jnp.exp(masked - m)
    gates = (ex / jnp.sum(ex, axis=-1, keepdims=True)) * sel
    denom = jnp.clip(jnp.sum(gates, axis=-1, keepdims=True), 1e-9, None)
    gates = gates / denom

    acc = None
    for e in range(_E):
        h = jnp.dot(x, W1_ref[e], preferred_element_type=jnp.float32)
        h = jnp.maximum(h + b1_ref[e:e + 1, :], 0.0)
        y = jnp.dot(h, W2_ref[e], preferred_element_type=jnp.float32)
        y = y + b2_ref[e:e + 1, :]
        gy = gates[:, e:e + 1] * y
        acc = gy if acc is None else acc + gy
    out_ref[...] = acc


def kernel(x, w_gate, w_noise, gate_threshold, experts_mask, noise, W1, b1, W2, b2):
    wgn = jnp.concatenate([w_gate, w_noise], axis=1)
    thr = gate_threshold.reshape(1, _E)
    mask = experts_mask.reshape(1, _E)

    grid = (_N // _BLK,)
    out = pl.pallas_call(
        _fused_body,
        grid=grid,
        in_specs=[
            pl.BlockSpec((_BLK, _D), lambda i: (i, 0)),
            pl.BlockSpec((_D, 2 * _E), lambda i: (0, 0)),
            pl.BlockSpec((1, _E), lambda i: (0, 0)),
            pl.BlockSpec((1, _E), lambda i: (0, 0)),
            pl.BlockSpec((_BLK, _E), lambda i: (i, 0)),
            pl.BlockSpec((_E, _D, _D), lambda i: (0, 0, 0)),
            pl.BlockSpec((_E, _D), lambda i: (0, 0)),
            pl.BlockSpec((_E, _D, _D), lambda i: (0, 0, 0)),
            pl.BlockSpec((_E, _D), lambda i: (0, 0)),
        ],
        out_specs=pl.BlockSpec((_BLK, _D), lambda i: (i, 0)),
        out_shape=jax.ShapeDtypeStruct((_N, _D), jnp.float32),
        compiler_params=pltpu.CompilerParams(
            dimension_semantics=("arbitrary",),
        ),
    )(x, wgn, thr, mask, noise, W1, b1, W2, b2)
    return out
